# Initial kernel scaffold; baseline (speedup 1.0000x reference)
import jax
import jax.numpy as jnp
from jax.experimental import pallas as pl


def kernel(action_indices, embedding_table):
    raise NotImplementedError("write your pallas kernel here")


# speedup vs baseline: 5.1043x; 5.1043x over previous
import functools

import jax
import jax.numpy as jnp
from jax import lax
from jax.experimental import pallas as pl
from jax.experimental.pallas import tpu as pltpu
from jax.experimental.pallas import tpu_sc as plsc

V = 52
D = 64
B2 = 16384 * 50 // 2
W = 2 * D
NC, NS = 2, 16
NW = NC * NS
IB = 128
SUB = 2
CHUNK = SUB * IB
B_PER_W = B2 // NW
N_CHUNKS = B_PER_W // CHUNK

_mesh = plsc.VectorSubcoreMesh(core_axis_name="c", subcore_axis_name="s")


@functools.partial(
    pl.kernel,
    out_type=jax.ShapeDtypeStruct((B2 // IB, IB, W), jnp.float32),
    mesh=_mesh,
    scratch_types=[
        pltpu.VMEM((SUB, IB), jnp.int32),
        pltpu.VMEM((SUB, IB, W), jnp.float32),
        pltpu.SemaphoreType.DMA,
    ],
)
def _gather_kernel(idx_hbm, table_hbm, out_hbm, idx_v, rows_v, sem):
    wid = lax.axis_index("s") * NC + lax.axis_index("c")
    base = wid * (B_PER_W // IB)

    def body(i, carry):
        blk = pl.multiple_of(base + i * SUB, SUB)
        pltpu.sync_copy(idx_hbm.at[pl.ds(blk, SUB)], idx_v)
        for j in range(SUB):
            pltpu.async_copy(table_hbm.at[idx_v.at[j]], rows_v.at[j], sem)
        for j in range(SUB):
            pltpu.make_async_copy(table_hbm.at[idx_v.at[j]], rows_v.at[j],
                                  sem).wait()
        pltpu.sync_copy(rows_v, out_hbm.at[pl.ds(blk, SUB)])
        return carry

    lax.fori_loop(0, N_CHUNKS, body, 0)


def kernel(action_indices, embedding_table):
    flat_idx = action_indices.reshape(-1).astype(jnp.int32)
    pair_idx = (flat_idx[0::2] * V + flat_idx[1::2]).reshape(B2 // IB, IB)
    pair_table = jnp.concatenate(
        [jnp.broadcast_to(embedding_table[:, None, :], (V, V, D)),
         jnp.broadcast_to(embedding_table[None, :, :], (V, V, D))],
        axis=-1).reshape(V * V, W)
    out = _gather_kernel(pair_idx, pair_table)
    return out.reshape(16384, 50, D)


# speedup vs baseline: 5.3257x; 1.0434x over previous
import functools

import jax
import jax.numpy as jnp
from jax import lax
from jax.experimental import pallas as pl
from jax.experimental.pallas import tpu as pltpu
from jax.experimental.pallas import tpu_sc as plsc

V = 52
D = 64
B2 = 16384 * 50 // 2
W = 2 * D
NC, NS = 2, 16
NW = NC * NS
IB = 128
SUB = 2
B_PER_W = B2 // NW
NBLK = B_PER_W // IB
N_CHUNKS = NBLK // SUB

_mesh = plsc.VectorSubcoreMesh(core_axis_name="c", subcore_axis_name="s")


@functools.partial(
    pl.kernel,
    out_type=jax.ShapeDtypeStruct((B2 // IB, IB, W), jnp.float32),
    mesh=_mesh,
    scratch_types=[
        pltpu.VMEM((SUB, IB), jnp.int32),
        pltpu.VMEM((SUB, IB), jnp.int32),
        pltpu.VMEM((SUB, IB, W), jnp.float32),
        pltpu.VMEM((SUB, IB, W), jnp.float32),
        pltpu.SemaphoreType.DMA,
        pltpu.SemaphoreType.DMA,
        pltpu.SemaphoreType.DMA,
        pltpu.SemaphoreType.DMA,
        pltpu.SemaphoreType.DMA,
        pltpu.SemaphoreType.DMA,
    ],
)
def _gather_kernel(idx_hbm, table_hbm, out_hbm, idx0, idx1,
                   rows0, rows1, isem0, isem1, gsem0, gsem1, ssem0, ssem1):
    wid = lax.axis_index("s") * NC + lax.axis_index("c")
    base = wid * NBLK
    idx = (idx0, idx1)
    rows = (rows0, rows1)
    isem = (isem0, isem1)
    gsem = (gsem0, gsem1)
    ssem = (ssem0, ssem1)

    def idx_load(c, b):
        blk = pl.multiple_of(base + c * SUB, SUB)
        return pltpu.make_async_copy(idx_hbm.at[pl.ds(blk, SUB)], idx[b],
                                     isem[b])

    def fire_gathers(b):
        for j in range(SUB):
            pltpu.async_copy(table_hbm.at[idx[b].at[j]], rows[b].at[j],
                             gsem[b])

    def drain_gathers(b):
        for j in range(SUB):
            pltpu.make_async_copy(table_hbm.at[idx[b].at[j]],
                                  rows[b].at[j], gsem[b]).wait()

    def store(c, b):
        blk = pl.multiple_of(base + c * SUB, SUB)
        return pltpu.make_async_copy(rows[b], out_hbm.at[pl.ds(blk, SUB)],
                                     ssem[b])

    idx_load(0, 0).start()
    idx_load(1, 1).start()
    idx_load(0, 0).wait()
    fire_gathers(0)

    def body(t, carry):
        for b in range(2):
            c = 2 * t + b
            other = 1 - b
            drain_gathers(b)

            @pl.when(c > 0)
            def _():
                store(c - 1, other).wait()

            @pl.when(c + 1 < N_CHUNKS)
            def _():
                idx_load(c + 1, other).wait()
                fire_gathers(other)

            store(c, b).start()

            @pl.when(c + 2 < N_CHUNKS)
            def _():
                idx_load(c + 2, b).start()
        return carry

    lax.fori_loop(0, N_CHUNKS // 2, body, 0)
    store(N_CHUNKS - 1, 1).wait()


def kernel(action_indices, embedding_table):
    flat_idx = action_indices.reshape(-1).astype(jnp.int32)
    pair_idx = (flat_idx[0::2] * V + flat_idx[1::2]).reshape(B2 // IB, IB)
    pair_table = jnp.concatenate(
        [jnp.broadcast_to(embedding_table[:, None, :], (V, V, D)),
         jnp.broadcast_to(embedding_table[None, :, :], (V, V, D))],
        axis=-1).reshape(V * V, W)
    out = _gather_kernel(pair_idx, pair_table)
    return out.reshape(16384, 50, D)
